# baseline (speedup 1.0000x reference)
import jax
import jax.numpy as jnp
from jax import lax
from jax.experimental import pallas as pl
from jax.experimental.pallas import tpu as pltpu
from jax.experimental.pallas import tpu_sc as plsc

_VOCAB = 1000000
_EMBED = 64
_B = 16384
_L = 50

_NC = 2
_NS = 16
_NW = _NC * _NS
_BW = _B // _NW
_BLK = 128
_K = _BW // _BLK


def _body(tok_hbm, table_hbm, out_hbm, idx_v, rows_v, gs0, gs1, os0, os1):
  gsem = (gs0, gs1)
  osem = (os0, os1)
  wid = lax.axis_index("s") * _NC + lax.axis_index("c")
  r0 = _K * wid

  def load_idx(l, s):
    pltpu.sync_copy(tok_hbm.at[l, pl.ds(r0, _K)], idx_v.at[s])

  def fire_gathers(s):
    for j in range(_K):
      pltpu.async_copy(table_hbm.at[idx_v.at[s, j]], rows_v.at[s, j], gsem[s])

  def drain_gathers(s):
    for j in range(_K):
      pltpu.make_async_copy(
          table_hbm.at[idx_v.at[s, j]], rows_v.at[s, j], gsem[s]).wait()

  def out_slice(l):
    return out_hbm.at[pl.ds(r0, _K), :, l]

  def fire_write(l, s):
    pltpu.async_copy(rows_v.at[s], out_slice(l), osem[s])

  def drain_write(l, s):
    pltpu.make_async_copy(rows_v.at[s], out_slice(l), osem[s]).wait()

  def step(l, s, drainw):
    if drainw:
      drain_write(l - 2, s)
    load_idx(l, s)
    fire_gathers(s)
    drain_gathers(1 - s)
    fire_write(l - 1, 1 - s)

  load_idx(0, 0)
  fire_gathers(0)
  step(1, 1, drainw=False)

  @pl.loop(2, _L - 2, step=2)
  def _(lbase):
    for b in range(2):
      step(lbase + b, b, drainw=True)

  step(_L - 2, 0, drainw=True)
  step(_L - 1, 1, drainw=True)
  drain_gathers(1)
  fire_write(_L - 1, 1)
  drain_write(_L - 2, 0)
  drain_write(_L - 1, 1)


@jax.jit
def _embed(tok3, table):
  mesh = plsc.VectorSubcoreMesh(core_axis_name="c", subcore_axis_name="s")
  return pl.kernel(
      _body,
      out_type=jax.ShapeDtypeStruct((_B // _BLK, _BLK, _L, _EMBED),
                                    jnp.float32),
      mesh=mesh,
      compiler_params=pltpu.CompilerParams(use_tc_tiling_on_sc=False),
      scratch_types=[
          pltpu.VMEM((2, _K, _BLK), jnp.int32),
          pltpu.VMEM((2, _K, _BLK, _EMBED), jnp.float32),
          pltpu.SemaphoreType.DMA,
          pltpu.SemaphoreType.DMA,
          pltpu.SemaphoreType.DMA,
          pltpu.SemaphoreType.DMA,
      ],
  )(tok3, table)


def kernel(tokens, table):
  tok3 = tokens.T.astype(jnp.int32).reshape(_L, _B // _BLK, _BLK)
  out = _embed(tok3, table)
  return out.reshape(_B, _L, _EMBED)
